# initial kernel scaffold (unmeasured)
import jax
import jax.numpy as jnp
from jax import lax
from jax.experimental import pallas as pl
from jax.experimental.pallas import tpu as pltpu

N_DEV = 32
B = 2
SQ = 128
DMODEL = 512
H_LOC = 4
DH = 64
ROWS = B * SQ
CHUNK = ROWS // N_DEV


def kernel(x, Wq, K_ext, V_ext, Wo):
    my = lax.axis_index("i")
    k_loc = lax.dynamic_slice_in_dim(K_ext, my * H_LOC, H_LOC, axis=2)
    v_loc = lax.dynamic_slice_in_dim(V_ext, my * H_LOC, H_LOC, axis=2)

    def body(x_ref, wq_ref, k_ref, v_ref, wo_ref, out_ref,
             partial_ref, inbox_ref, acc_ref,
             send1, recv1, send2, recv2):
        me = lax.axis_index("i")

        xq = x_ref[...].reshape(ROWS, DMODEL)
        q_all = jnp.dot(xq, wq_ref[...], preferred_element_type=jnp.float32)
        ctx_rows = []
        for b in range(B):
            head_cols = []
            for h in range(H_LOC):
                q = q_all[b * SQ:(b + 1) * SQ, h * DH:(h + 1) * DH]
                k = k_ref[b, :, h, :]
                v = v_ref[b, :, h, :]
                s = lax.dot_general(
                    q, k, (((1,), (1,)), ((), ())),
                    preferred_element_type=jnp.float32,
                ) * 0.125
                s = s - jnp.max(s, axis=-1, keepdims=True)
                w = jnp.exp(s)
                w = w / jnp.sum(w, axis=-1, keepdims=True)
                head_cols.append(
                    jnp.dot(w, v, preferred_element_type=jnp.float32))
            ctx_rows.append(jnp.concatenate(head_cols, axis=1))
        ctx = jnp.concatenate(ctx_rows, axis=0)
        partial = jnp.dot(ctx, wo_ref[...],
                          preferred_element_type=jnp.float32)
        partial_ref[...] = partial.reshape(N_DEV, CHUNK, DMODEL)
        inbox_ref[0:1] = partial_ref[pl.ds(me, 1)]

        sends1 = []
        for d in range(1, N_DEV):
            peer = (me + d) % N_DEV
            slot = N_DEV - d
            rdma = pltpu.make_async_remote_copy(
                src_ref=partial_ref.at[pl.ds(peer, 1)],
                dst_ref=inbox_ref.at[pl.ds(slot, 1)],
                send_sem=send1.at[d],
                recv_sem=recv1.at[slot],
                device_id=(peer,),
                device_id_type=pl.DeviceIdType.MESH,
            )
            rdma.start()
            sends1.append(rdma)
        for d in range(1, N_DEV):
            pltpu.make_async_remote_copy(
                src_ref=inbox_ref.at[pl.ds(d, 1)],
                dst_ref=inbox_ref.at[pl.ds(d, 1)],
                send_sem=send1.at[d],
                recv_sem=recv1.at[d],
                device_id=(me,),
                device_id_type=pl.DeviceIdType.MESH,
            ).wait_recv()
        for rdma in sends1:
            rdma.wait_send()

        acc_ref[...] = jnp.sum(inbox_ref[...], axis=0, keepdims=True)

        out_ref[pl.ds(me, 1)] = acc_ref[...]
        sends2 = []
        for d in range(1, N_DEV):
            peer = (me + d) % N_DEV
            slot = N_DEV - d
            rdma = pltpu.make_async_remote_copy(
                src_ref=acc_ref,
                dst_ref=out_ref.at[pl.ds(me, 1)],
                send_sem=send2.at[d],
                recv_sem=recv2.at[slot],
                device_id=(peer,),
                device_id_type=pl.DeviceIdType.MESH,
            )
            rdma.start()
            sends2.append(rdma)
        for d in range(1, N_DEV):
            pltpu.make_async_remote_copy(
                src_ref=out_ref.at[pl.ds(d, 1)],
                dst_ref=out_ref.at[pl.ds(d, 1)],
                send_sem=send2.at[d],
                recv_sem=recv2.at[d],
                device_id=(me,),
                device_id_type=pl.DeviceIdType.MESH,
            ).wait_recv()
        for rdma in sends2:
            rdma.wait_send()

    out = pl.pallas_call(
        body,
        out_shape=jax.ShapeDtypeStruct((N_DEV, CHUNK, DMODEL), jnp.float32),
        in_specs=[pl.BlockSpec(memory_space=pltpu.VMEM)] * 5,
        out_specs=pl.BlockSpec(memory_space=pltpu.VMEM),
        scratch_shapes=[
            pltpu.VMEM((N_DEV, CHUNK, DMODEL), jnp.float32),
            pltpu.VMEM((N_DEV, CHUNK, DMODEL), jnp.float32),
            pltpu.VMEM((1, CHUNK, DMODEL), jnp.float32),
            pltpu.SemaphoreType.DMA((N_DEV,)),
            pltpu.SemaphoreType.DMA((N_DEV,)),
            pltpu.SemaphoreType.DMA((N_DEV,)),
            pltpu.SemaphoreType.DMA((N_DEV,)),
        ],
        compiler_params=pltpu.CompilerParams(collective_id=0),
    )(x, Wq, k_loc, v_loc, Wo)
    return out.reshape(B, SQ, DMODEL)


# baseline (device time: 48335 ns/iter reference)
import jax
import jax.numpy as jnp
from jax import lax
from jax.experimental import pallas as pl
from jax.experimental.pallas import tpu as pltpu

N_DEV = 32
B = 2
SQ = 128
DMODEL = 512
H_LOC = 4
DH = 64
ROWS = B * SQ
CHUNK = ROWS // N_DEV


def kernel(x, Wq, K_ext, V_ext, Wo):
    my = lax.axis_index("i")
    k_loc = lax.dynamic_slice_in_dim(K_ext, my * H_LOC, H_LOC, axis=2)
    v_loc = lax.dynamic_slice_in_dim(V_ext, my * H_LOC, H_LOC, axis=2)

    def body(x_ref, wq_ref, k_ref, v_ref, wo_ref, out_ref,
             partial_ref, inbox_ref, acc_ref,
             send1, recv1, send2, recv2):
        me = lax.axis_index("i")

        xq = x_ref[...].reshape(ROWS, DMODEL)
        q_all = jnp.dot(xq, wq_ref[...], preferred_element_type=jnp.float32)
        ctx_rows = []
        for b in range(B):
            head_cols = []
            for h in range(H_LOC):
                q = q_all[b * SQ:(b + 1) * SQ, h * DH:(h + 1) * DH]
                k = k_ref[b, :, h, :]
                v = v_ref[b, :, h, :]
                s = lax.dot_general(
                    q, k, (((1,), (1,)), ((), ())),
                    preferred_element_type=jnp.float32,
                ) * 0.125
                s = s - jnp.max(s, axis=-1, keepdims=True)
                w = jnp.exp(s)
                w = w / jnp.sum(w, axis=-1, keepdims=True)
                head_cols.append(
                    jnp.dot(w, v, preferred_element_type=jnp.float32))
            ctx_rows.append(jnp.concatenate(head_cols, axis=1))
        ctx = jnp.concatenate(ctx_rows, axis=0)
        partial = jnp.dot(ctx, wo_ref[...],
                          preferred_element_type=jnp.float32)
        partial_ref[...] = partial.reshape(N_DEV, CHUNK, DMODEL)
        inbox_ref[0:1] = partial_ref[pl.ds(me, 1)]

        sends1 = []
        for d in range(1, N_DEV):
            peer = (me + d) % N_DEV
            slot = N_DEV - d
            rdma = pltpu.make_async_remote_copy(
                src_ref=partial_ref.at[pl.ds(peer, 1)],
                dst_ref=inbox_ref.at[pl.ds(slot, 1)],
                send_sem=send1.at[d],
                recv_sem=recv1.at[slot],
                device_id=(peer,),
                device_id_type=pl.DeviceIdType.MESH,
            )
            rdma.start()
            sends1.append(rdma)
        for d in range(1, N_DEV):
            pltpu.make_async_remote_copy(
                src_ref=inbox_ref.at[pl.ds(d, 1)],
                dst_ref=inbox_ref.at[pl.ds(d, 1)],
                send_sem=send1.at[d],
                recv_sem=recv1.at[d],
                device_id=(me,),
                device_id_type=pl.DeviceIdType.MESH,
            ).wait_recv()
        for rdma in sends1:
            rdma.wait_send()

        acc_ref[...] = jnp.sum(inbox_ref[...], axis=0, keepdims=True)

        out_ref[pl.ds(me, 1)] = acc_ref[...]
        sends2 = []
        for d in range(1, N_DEV):
            peer = (me + d) % N_DEV
            slot = N_DEV - d
            rdma = pltpu.make_async_remote_copy(
                src_ref=acc_ref,
                dst_ref=out_ref.at[pl.ds(me, 1)],
                send_sem=send2.at[d],
                recv_sem=recv2.at[slot],
                device_id=(peer,),
                device_id_type=pl.DeviceIdType.MESH,
            )
            rdma.start()
            sends2.append(rdma)
        for d in range(1, N_DEV):
            pltpu.make_async_remote_copy(
                src_ref=out_ref.at[pl.ds(d, 1)],
                dst_ref=out_ref.at[pl.ds(d, 1)],
                send_sem=send2.at[d],
                recv_sem=recv2.at[d],
                device_id=(me,),
                device_id_type=pl.DeviceIdType.MESH,
            ).wait_recv()
        for rdma in sends2:
            rdma.wait_send()

    out = pl.pallas_call(
        body,
        out_shape=jax.ShapeDtypeStruct((N_DEV, CHUNK, DMODEL), jnp.float32),
        in_specs=[pl.BlockSpec(memory_space=pltpu.VMEM)] * 5,
        out_specs=pl.BlockSpec(memory_space=pltpu.VMEM),
        scratch_shapes=[
            pltpu.VMEM((N_DEV, CHUNK, DMODEL), jnp.float32),
            pltpu.VMEM((N_DEV, CHUNK, DMODEL), jnp.float32),
            pltpu.VMEM((1, CHUNK, DMODEL), jnp.float32),
            pltpu.SemaphoreType.DMA((N_DEV,)),
            pltpu.SemaphoreType.DMA((N_DEV,)),
            pltpu.SemaphoreType.DMA((N_DEV,)),
            pltpu.SemaphoreType.DMA((N_DEV,)),
        ],
    )(x, Wq, k_loc, v_loc, Wo)
    return out.reshape(B, SQ, DMODEL)
